# Initial kernel scaffold; baseline (speedup 1.0000x reference)
import jax
import jax.numpy as jnp
from jax.experimental import pallas as pl


def kernel(x, input_node_ids, edge_index, edge_attr, seed_ctx_feat, seed_lookup_idx, embed_table, W_feat, b_feat, W_edge, b_edge, W1, W1s, b1, W2, W2s, b2, W_ctx, b_ctx, W_pred, b_pred):
    raise NotImplementedError("write your pallas kernel here")


# speedup vs baseline: 3.4013x; 3.4013x over previous
import functools

import jax
import jax.numpy as jnp
from jax import lax
from jax.experimental import pallas as pl
from jax.experimental.pallas import tpu as pltpu
from jax.experimental.pallas import tpu_sc as plsc

F32 = jnp.float32
LANES = 128


def _tc_node_encode(x, W_feat, b_feat, num_seeds):
    N, DF = x.shape
    DH = W_feat.shape[1]
    BM = 2000

    def body(x_ref, w_ref, b_ref, o_ref):
        i = pl.program_id(0)
        h = jnp.dot(x_ref[...], w_ref[...], preferred_element_type=F32)
        h = h + b_ref[...]
        row = i * BM + lax.broadcasted_iota(jnp.int32, (BM, 1), 0)
        o_ref[...] = jnp.where(row < num_seeds, 0.0, h)

    return pl.pallas_call(
        body,
        grid=(N // BM,),
        in_specs=[
            pl.BlockSpec((BM, DF), lambda i: (i, 0)),
            pl.BlockSpec((DF, DH), lambda i: (0, 0)),
            pl.BlockSpec((1, DH), lambda i: (0, 0)),
        ],
        out_specs=pl.BlockSpec((BM, DH), lambda i: (i, 0)),
        out_shape=jax.ShapeDtypeStruct((N, DH), F32),
    )(x, W_feat, b_feat.reshape(1, DH))


def _tc_edge_encode(edge_attr, W_edge, b_edge):
    E, DE = edge_attr.shape
    BE = 8000

    def body(e_ref, w_ref, b_ref, o_ref):
        h = jnp.dot(e_ref[...], w_ref[...], preferred_element_type=F32)
        o_ref[...] = jnp.maximum(h + b_ref[...], 0.0)

    return pl.pallas_call(
        body,
        grid=(E // BE,),
        in_specs=[
            pl.BlockSpec((BE, DE), lambda i: (i, 0)),
            pl.BlockSpec((DE, DE), lambda i: (0, 0)),
            pl.BlockSpec((1, DE), lambda i: (0, 0)),
        ],
        out_specs=pl.BlockSpec((BE, DE), lambda i: (i, 0)),
        out_shape=jax.ShapeDtypeStruct((E, DE), F32),
    )(edge_attr, W_edge, b_edge.reshape(1, DE))


def _tc_layer1_pre(H_feat, H_id, ed0, ed1, W1, W1s, W2, b1, DID, DE):
    N, DH = H_feat.shape
    BM = 2000

    def body(hf, hid, e0r, e1r, w1, w1s, w2, b1r, p1_o, base1_o, egg2_o):
        ed = e0r[...] + e1r[...]
        eagg = ed[:, 0:DE]
        deg = jnp.maximum(ed[:, DE:DE + 1], 1.0)
        es = eagg / deg
        w1v = w1[...]
        w1sv = w1s[...]
        hfv = hf[...]
        hidv = hid[...][:, 0:DID]
        p1_o[...] = (jnp.dot(hfv, w1v[0:DH], preferred_element_type=F32)
                     + jnp.dot(hidv, w1v[DH:DH + DID], preferred_element_type=F32))
        base1_o[...] = (jnp.dot(hfv, w1sv[0:DH], preferred_element_type=F32)
                        + jnp.dot(hidv, w1sv[DH:DH + DID], preferred_element_type=F32)
                        + jnp.dot(es, w1v[DH + DID:DH + DID + DE], preferred_element_type=F32)
                        + b1r[...])
        egg2_o[...] = jnp.dot(es, w2[...][DH:DH + DE], preferred_element_type=F32)

    spec_h = pl.BlockSpec((BM, DH), lambda i: (i, 0))
    spec_l = pl.BlockSpec((BM, LANES), lambda i: (i, 0))
    return pl.pallas_call(
        body,
        grid=(N // BM,),
        in_specs=[
            spec_h, spec_l, spec_l, spec_l,
            pl.BlockSpec(W1.shape, lambda i: (0, 0)),
            pl.BlockSpec(W1s.shape, lambda i: (0, 0)),
            pl.BlockSpec(W2.shape, lambda i: (0, 0)),
            pl.BlockSpec((1, DH), lambda i: (0, 0)),
        ],
        out_specs=[spec_h, spec_h, spec_h],
        out_shape=[jax.ShapeDtypeStruct((N, DH), F32)] * 3,
    )(H_feat, H_id, ed0, ed1, W1, W1s, W2, b1.reshape(1, DH))


def _tc_layer2_pre(a0, a1, ed0, ed1, base1, egg2, W2, W2s, b2, DE):
    N, DH = a0.shape
    BM = 2000

    def body(a0r, a1r, e0r, e1r, b1r, eg, w2, w2s, b2r, p2_o, base2_o):
        deg = jnp.maximum(e0r[...][:, DE:DE + 1] + e1r[...][:, DE:DE + 1], 1.0)
        h1 = jnp.maximum((a0r[...] + a1r[...]) / deg + b1r[...], 0.0)
        p2_o[...] = jnp.dot(h1, w2[...][0:DH], preferred_element_type=F32)
        base2_o[...] = (jnp.dot(h1, w2s[...], preferred_element_type=F32)
                        + eg[...] + b2r[...])

    spec_h = pl.BlockSpec((BM, DH), lambda i: (i, 0))
    spec_l = pl.BlockSpec((BM, LANES), lambda i: (i, 0))
    return pl.pallas_call(
        body,
        grid=(N // BM,),
        in_specs=[
            spec_h, spec_h, spec_l, spec_l, spec_h, spec_h,
            pl.BlockSpec(W2.shape, lambda i: (0, 0)),
            pl.BlockSpec(W2s.shape, lambda i: (0, 0)),
            pl.BlockSpec((1, DH), lambda i: (0, 0)),
        ],
        out_specs=[spec_h, spec_h],
        out_shape=[jax.ShapeDtypeStruct((N, DH), F32)] * 2,
    )(a0, a1, ed0, ed1, base1, egg2, W2, W2s, b2.reshape(1, DH))


def _tc_layer2_post(a0, a1, ed0, ed1, base2, DE):
    N, DH = a0.shape
    BM = 2000

    def body(a0r, a1r, e0r, e1r, b2r, h2_o):
        deg = jnp.maximum(e0r[...][:, DE:DE + 1] + e1r[...][:, DE:DE + 1], 1.0)
        h2_o[...] = jnp.maximum((a0r[...] + a1r[...]) / deg + b2r[...], 0.0)

    spec_h = pl.BlockSpec((BM, DH), lambda i: (i, 0))
    spec_l = pl.BlockSpec((BM, LANES), lambda i: (i, 0))
    return pl.pallas_call(
        body,
        grid=(N // BM,),
        in_specs=[spec_h, spec_h, spec_l, spec_l, spec_h],
        out_specs=spec_h,
        out_shape=jax.ShapeDtypeStruct((N, DH), F32),
    )(a0, a1, ed0, ed1, base2)


def _tc_predict(seed_embeds, seed_ctx_feat, W_ctx, b_ctx, W_pred, b_pred):
    S, DH = seed_embeds.shape
    DCTX = seed_ctx_feat.shape[1]
    OUT = W_pred.shape[1]

    def body(se, ctx, wc, bc, wp, bp, o_ref):
        ctxe = jnp.maximum(
            jnp.dot(ctx[...], wc[...], preferred_element_type=F32) + bc[...], 0.0)
        wpv = wp[...]
        o_ref[...] = (jnp.dot(se[...], wpv[0:DH], preferred_element_type=F32)
                      + jnp.dot(ctxe, wpv[DH:DH + DCTX], preferred_element_type=F32)
                      + bp[...])

    return pl.pallas_call(
        body,
        out_shape=jax.ShapeDtypeStruct((S, OUT), F32),
    )(seed_embeds, seed_ctx_feat, W_ctx, b_ctx.reshape(1, DCTX),
      W_pred, b_pred.reshape(1, OUT))


_NW = 32
_NC = 2


def _sc_encode_pass(embed_pad, node_ids, H_e, dst, zeros_l):
    N = embed_pad.shape[0]
    E = dst.shape[0]
    DE = H_e.shape[1]
    CH = 80
    EPW = E // _NW
    NCHE = EPW // CH
    NCHN = N // CH
    NCHN_PW = (NCHN + _NW - 1) // _NW
    RPT = (N // 16) // 8 * 8
    TAIL = N - 16 * RPT

    mesh = plsc.VectorSubcoreMesh(core_axis_name="c", subcore_axis_name="s")

    @functools.partial(
        pl.kernel,
        out_type=[
            jax.ShapeDtypeStruct((N, LANES), F32),
            jax.ShapeDtypeStruct((_NC, N, LANES), F32),
        ],
        mesh=mesh,
        scratch_types=[
            pltpu.VMEM((CH,), jnp.int32),
            pltpu.VMEM((CH, DE), F32),
            pltpu.VMEM((CH, LANES), F32),
            pltpu.VMEM((CH, LANES), F32),
            pltpu.VMEM_SHARED((N, LANES), F32),
            pltpu.SemaphoreType.DMA,
        ],
    )
    def k(emb_hbm, ids_hbm, he_hbm, dst_hbm, z_hbm,
          hid_out, ed_out,
          idx_v, he_v, rows_v, grows_v, acc, sem):
        cid = lax.axis_index("c")
        sid = lax.axis_index("s")
        wid = sid * _NC + cid

        pltpu.sync_copy(z_hbm.at[pl.ds(0, CH)], rows_v)
        one_hot = jnp.where(lax.iota(jnp.int32, 16) == 0, 1.0, 0.0).astype(F32)

        def fill_ones(i, _):
            rows_v[i, pl.ds(DE, 16)] = one_hot
            return _
        lax.fori_loop(0, CH, fill_ones, None)

        pltpu.sync_copy(z_hbm.at[pl.ds(sid * RPT, RPT)],
                        acc.at[pl.ds(sid * RPT, RPT)])

        @pl.when(sid == 0)
        def _():
            pltpu.sync_copy(z_hbm.at[pl.ds(16 * RPT, TAIL)],
                            acc.at[pl.ds(16 * RPT, TAIL)])
        plsc.subcore_barrier()

        def nbody(i, _):
            c = wid + _NW * i

            @pl.when(c < NCHN)
            def _():
                off = c * CH
                pltpu.sync_copy(ids_hbm.at[pl.ds(off, CH)], idx_v)
                pltpu.async_copy(emb_hbm.at[idx_v], grows_v, sem).wait()
                pltpu.sync_copy(grows_v, hid_out.at[pl.ds(off, CH)])
            return _
        lax.fori_loop(0, NCHN_PW, nbody, None)

        ebase = wid * EPW

        def ebody(i, _):
            off = ebase + i * CH
            pltpu.sync_copy(dst_hbm.at[pl.ds(off, CH)], idx_v)
            pltpu.sync_copy(he_hbm.at[pl.ds(off, CH)], he_v)

            def widen(r, _):
                rows_v[r, pl.ds(0, 16)] = he_v[r, :]
                return _
            lax.fori_loop(0, CH, widen, None)
            pltpu.sync_copy(rows_v, acc.at[idx_v], add=True)
            return _
        lax.fori_loop(0, NCHE, ebody, None)
        plsc.subcore_barrier()

        pltpu.sync_copy(acc.at[pl.ds(sid * RPT, RPT)],
                        ed_out.at[cid, pl.ds(sid * RPT, RPT)])

        @pl.when(sid == 0)
        def _():
            pltpu.sync_copy(acc.at[pl.ds(16 * RPT, TAIL)],
                            ed_out.at[cid, pl.ds(16 * RPT, TAIL)])

    return k(embed_pad, node_ids, H_e, dst, zeros_l)


def _sc_gather_scatter(P, src, dst, zeros_l):
    N, DH = P.shape
    E = src.shape[0]
    CH = 80
    EPW = E // _NW
    NCH = EPW // CH
    RPT = (N // 16) // 8 * 8
    TAIL = N - 16 * RPT

    mesh = plsc.VectorSubcoreMesh(core_axis_name="c", subcore_axis_name="s")

    @functools.partial(
        pl.kernel,
        out_type=jax.ShapeDtypeStruct((_NC, N, DH), F32),
        mesh=mesh,
        scratch_types=[
            pltpu.VMEM((CH,), jnp.int32),
            pltpu.VMEM((CH,), jnp.int32),
            pltpu.VMEM((CH, DH), F32),
            pltpu.VMEM_SHARED((N, DH), F32),
            pltpu.SemaphoreType.DMA,
        ],
    )
    def k(p_hbm, src_hbm, dst_hbm, z_hbm, out_hbm,
          sidx_v, didx_v, rows_v, acc, sem):
        cid = lax.axis_index("c")
        sid = lax.axis_index("s")
        wid = sid * _NC + cid

        pltpu.sync_copy(z_hbm.at[pl.ds(sid * RPT, RPT)],
                        acc.at[pl.ds(sid * RPT, RPT)])

        @pl.when(sid == 0)
        def _():
            pltpu.sync_copy(z_hbm.at[pl.ds(16 * RPT, TAIL)],
                            acc.at[pl.ds(16 * RPT, TAIL)])
        plsc.subcore_barrier()

        base = wid * EPW

        def body(i, _):
            off = base + i * CH
            pltpu.sync_copy(src_hbm.at[pl.ds(off, CH)], sidx_v)
            pltpu.sync_copy(dst_hbm.at[pl.ds(off, CH)], didx_v)
            pltpu.async_copy(p_hbm.at[sidx_v], rows_v, sem).wait()
            pltpu.sync_copy(rows_v, acc.at[didx_v], add=True)
            return _
        lax.fori_loop(0, NCH, body, None)
        plsc.subcore_barrier()

        pltpu.sync_copy(acc.at[pl.ds(sid * RPT, RPT)],
                        out_hbm.at[cid, pl.ds(sid * RPT, RPT)])

        @pl.when(sid == 0)
        def _():
            pltpu.sync_copy(acc.at[pl.ds(16 * RPT, TAIL)],
                            out_hbm.at[cid, pl.ds(16 * RPT, TAIL)])

    return k(P, src, dst, zeros_l)


def _sc_seed_gather(H2, seed_idx):
    N, DH = H2.shape
    S = seed_idx.shape[0]
    BPW = S // _NW

    mesh = plsc.VectorSubcoreMesh(core_axis_name="c", subcore_axis_name="s")

    @functools.partial(
        pl.kernel,
        out_type=jax.ShapeDtypeStruct((S, DH), F32),
        mesh=mesh,
        scratch_types=[
            pltpu.VMEM((BPW,), jnp.int32),
            pltpu.VMEM((BPW, DH), F32),
            pltpu.SemaphoreType.DMA,
        ],
    )
    def k(h_hbm, idx_hbm, out_hbm, idx_v, rows_v, sem):
        wid = lax.axis_index("s") * _NC + lax.axis_index("c")
        base = wid * BPW
        pltpu.sync_copy(idx_hbm.at[pl.ds(base, BPW)], idx_v)
        pltpu.async_copy(h_hbm.at[idx_v], rows_v, sem).wait()
        pltpu.sync_copy(rows_v, out_hbm.at[pl.ds(base, BPW)])

    return k(H2, seed_idx)


def kernel(x, input_node_ids, edge_index, edge_attr, seed_ctx_feat,
           seed_lookup_idx, embed_table, W_feat, b_feat, W_edge, b_edge,
           W1, W1s, b1, W2, W2s, b2, W_ctx, b_ctx, W_pred, b_pred):
    N = x.shape[0]
    S = seed_ctx_feat.shape[0]
    DE = edge_attr.shape[1]
    DID = embed_table.shape[1]

    src = edge_index[0].astype(jnp.int32)
    dst = edge_index[1].astype(jnp.int32)
    ids = input_node_ids.astype(jnp.int32)
    sidx = seed_lookup_idx.astype(jnp.int32)
    zeros_l = jnp.zeros((N, LANES), F32)
    embed_pad = jnp.pad(embed_table, ((0, 0), (0, LANES - DID)))

    H_feat = _tc_node_encode(x, W_feat, b_feat, S)
    H_e = _tc_edge_encode(edge_attr, W_edge, b_edge)

    H_id, edP = _sc_encode_pass(embed_pad, ids, H_e, dst, zeros_l)

    P1, base1, egg2 = _tc_layer1_pre(
        H_feat, H_id, edP[0], edP[1], W1, W1s, W2, b1, DID, DE)

    agg1 = _sc_gather_scatter(P1, src, dst, zeros_l)

    P2, base2 = _tc_layer2_pre(
        agg1[0], agg1[1], edP[0], edP[1], base1, egg2, W2, W2s, b2, DE)

    agg2 = _sc_gather_scatter(P2, src, dst, zeros_l)

    H2 = _tc_layer2_post(agg2[0], agg2[1], edP[0], edP[1], base2, DE)

    seed_embeds = _sc_seed_gather(H2, sidx)

    return _tc_predict(seed_embeds, seed_ctx_feat, W_ctx, b_ctx,
                       W_pred, b_pred)


# speedup vs baseline: 4.7456x; 1.3952x over previous
import functools

import jax
import jax.numpy as jnp
from jax import lax
from jax.experimental import pallas as pl
from jax.experimental.pallas import tpu as pltpu
from jax.experimental.pallas import tpu_sc as plsc

F32 = jnp.float32
LANES = 128


def _tc_node_encode(x, W_feat, b_feat, num_seeds):
    N, DF = x.shape
    DH = W_feat.shape[1]
    BM = 2000

    def body(x_ref, w_ref, b_ref, o_ref):
        i = pl.program_id(0)
        h = jnp.dot(x_ref[...], w_ref[...], preferred_element_type=F32)
        h = h + b_ref[...]
        row = i * BM + lax.broadcasted_iota(jnp.int32, (BM, 1), 0)
        o_ref[...] = jnp.where(row < num_seeds, 0.0, h)

    return pl.pallas_call(
        body,
        grid=(N // BM,),
        in_specs=[
            pl.BlockSpec((BM, DF), lambda i: (i, 0)),
            pl.BlockSpec((DF, DH), lambda i: (0, 0)),
            pl.BlockSpec((1, DH), lambda i: (0, 0)),
        ],
        out_specs=pl.BlockSpec((BM, DH), lambda i: (i, 0)),
        out_shape=jax.ShapeDtypeStruct((N, DH), F32),
    )(x, W_feat, b_feat.reshape(1, DH))


def _tc_edge_encode(edge_attr, W_edge, b_edge):
    E, DE = edge_attr.shape
    BE = 8000

    def body(e_ref, w_ref, b_ref, o_ref):
        h = jnp.dot(e_ref[...], w_ref[...], preferred_element_type=F32)
        o_ref[...] = jnp.maximum(h + b_ref[...], 0.0)

    return pl.pallas_call(
        body,
        grid=(E // BE,),
        in_specs=[
            pl.BlockSpec((BE, DE), lambda i: (i, 0)),
            pl.BlockSpec((DE, DE), lambda i: (0, 0)),
            pl.BlockSpec((1, DE), lambda i: (0, 0)),
        ],
        out_specs=pl.BlockSpec((BE, DE), lambda i: (i, 0)),
        out_shape=jax.ShapeDtypeStruct((E, DE), F32),
    )(edge_attr, W_edge, b_edge.reshape(1, DE))


def _tc_layer1_pre(H_feat, H_id, ed0, ed1, W1, W1s, W2, b1, DID, DE):
    N, DH = H_feat.shape
    BM = 2000

    def body(hf, hid, e0r, e1r, w1, w1s, w2, b1r, p1_o, base1_o, egg2_o):
        ed = e0r[...] + e1r[...]
        eagg = ed[:, 0:DE]
        deg = jnp.maximum(ed[:, DE:DE + 1], 1.0)
        es = eagg / deg
        w1v = w1[...]
        w1sv = w1s[...]
        hfv = hf[...]
        hidv = hid[...][:, 0:DID]
        p1_o[...] = (jnp.dot(hfv, w1v[0:DH], preferred_element_type=F32)
                     + jnp.dot(hidv, w1v[DH:DH + DID], preferred_element_type=F32))
        base1_o[...] = (jnp.dot(hfv, w1sv[0:DH], preferred_element_type=F32)
                        + jnp.dot(hidv, w1sv[DH:DH + DID], preferred_element_type=F32)
                        + jnp.dot(es, w1v[DH + DID:DH + DID + DE], preferred_element_type=F32)
                        + b1r[...])
        egg2_o[...] = jnp.dot(es, w2[...][DH:DH + DE], preferred_element_type=F32)

    spec_h = pl.BlockSpec((BM, DH), lambda i: (i, 0))
    spec_l = pl.BlockSpec((BM, LANES), lambda i: (i, 0))
    return pl.pallas_call(
        body,
        grid=(N // BM,),
        in_specs=[
            spec_h, spec_l, spec_l, spec_l,
            pl.BlockSpec(W1.shape, lambda i: (0, 0)),
            pl.BlockSpec(W1s.shape, lambda i: (0, 0)),
            pl.BlockSpec(W2.shape, lambda i: (0, 0)),
            pl.BlockSpec((1, DH), lambda i: (0, 0)),
        ],
        out_specs=[spec_h, spec_h, spec_h],
        out_shape=[jax.ShapeDtypeStruct((N, DH), F32)] * 3,
    )(H_feat, H_id, ed0, ed1, W1, W1s, W2, b1.reshape(1, DH))


def _tc_layer2_pre(a0, a1, ed0, ed1, base1, egg2, W2, W2s, b2, DE):
    N, DH = a0.shape
    BM = 2000

    def body(a0r, a1r, e0r, e1r, b1r, eg, w2, w2s, b2r, p2_o, base2_o):
        deg = jnp.maximum(e0r[...][:, DE:DE + 1] + e1r[...][:, DE:DE + 1], 1.0)
        h1 = jnp.maximum((a0r[...] + a1r[...]) / deg + b1r[...], 0.0)
        p2_o[...] = jnp.dot(h1, w2[...][0:DH], preferred_element_type=F32)
        base2_o[...] = (jnp.dot(h1, w2s[...], preferred_element_type=F32)
                        + eg[...] + b2r[...])

    spec_h = pl.BlockSpec((BM, DH), lambda i: (i, 0))
    spec_l = pl.BlockSpec((BM, LANES), lambda i: (i, 0))
    return pl.pallas_call(
        body,
        grid=(N // BM,),
        in_specs=[
            spec_h, spec_h, spec_l, spec_l, spec_h, spec_h,
            pl.BlockSpec(W2.shape, lambda i: (0, 0)),
            pl.BlockSpec(W2s.shape, lambda i: (0, 0)),
            pl.BlockSpec((1, DH), lambda i: (0, 0)),
        ],
        out_specs=[spec_h, spec_h],
        out_shape=[jax.ShapeDtypeStruct((N, DH), F32)] * 2,
    )(a0, a1, ed0, ed1, base1, egg2, W2, W2s, b2.reshape(1, DH))


def _tc_layer2_post(a0, a1, ed0, ed1, base2, DE):
    N, DH = a0.shape
    BM = 2000

    def body(a0r, a1r, e0r, e1r, b2r, h2_o):
        deg = jnp.maximum(e0r[...][:, DE:DE + 1] + e1r[...][:, DE:DE + 1], 1.0)
        h2_o[...] = jnp.maximum((a0r[...] + a1r[...]) / deg + b2r[...], 0.0)

    spec_h = pl.BlockSpec((BM, DH), lambda i: (i, 0))
    spec_l = pl.BlockSpec((BM, LANES), lambda i: (i, 0))
    return pl.pallas_call(
        body,
        grid=(N // BM,),
        in_specs=[spec_h, spec_h, spec_l, spec_l, spec_h],
        out_specs=spec_h,
        out_shape=jax.ShapeDtypeStruct((N, DH), F32),
    )(a0, a1, ed0, ed1, base2)


def _tc_predict(seed_embeds, seed_ctx_feat, W_ctx, b_ctx, W_pred, b_pred):
    S, DH = seed_embeds.shape
    DCTX = seed_ctx_feat.shape[1]
    OUT = W_pred.shape[1]

    def body(se, ctx, wc, bc, wp, bp, o_ref):
        ctxe = jnp.maximum(
            jnp.dot(ctx[...], wc[...], preferred_element_type=F32) + bc[...], 0.0)
        wpv = wp[...]
        o_ref[...] = (jnp.dot(se[...], wpv[0:DH], preferred_element_type=F32)
                      + jnp.dot(ctxe, wpv[DH:DH + DCTX], preferred_element_type=F32)
                      + bp[...])

    return pl.pallas_call(
        body,
        out_shape=jax.ShapeDtypeStruct((S, OUT), F32),
    )(seed_embeds, seed_ctx_feat, W_ctx, b_ctx.reshape(1, DCTX),
      W_pred, b_pred.reshape(1, OUT))


_NW = 32
_NC = 2


def _sc_encode_pass(embed_pad, node_ids, H_e, dst, zeros_l):
    N = embed_pad.shape[0]
    E = dst.shape[0]
    DE = H_e.shape[1]
    CH = 80
    EPW = E // _NW
    NCHE = EPW // CH
    NCHN = N // CH
    NCHN_PW = (NCHN + _NW - 1) // _NW
    RPT = (N // 16) // 8 * 8
    TAIL = N - 16 * RPT

    mesh = plsc.VectorSubcoreMesh(core_axis_name="c", subcore_axis_name="s")

    @functools.partial(
        pl.kernel,
        out_type=[
            jax.ShapeDtypeStruct((N, LANES), F32),
            jax.ShapeDtypeStruct((_NC, N, LANES), F32),
        ],
        mesh=mesh,
        scratch_types=[
            pltpu.VMEM((CH,), jnp.int32),
            pltpu.VMEM((CH, DE), F32),
            pltpu.VMEM((CH, LANES), F32),
            pltpu.VMEM((CH, LANES), F32),
            pltpu.VMEM_SHARED((N, LANES), F32),
            pltpu.SemaphoreType.DMA,
        ],
    )
    def k(emb_hbm, ids_hbm, he_hbm, dst_hbm, z_hbm,
          hid_out, ed_out,
          idx_v, he_v, rows_v, grows_v, acc, sem):
        cid = lax.axis_index("c")
        sid = lax.axis_index("s")
        wid = sid * _NC + cid

        pltpu.sync_copy(z_hbm.at[pl.ds(0, CH)], rows_v)
        one_hot = jnp.where(lax.iota(jnp.int32, 16) == 0, 1.0, 0.0).astype(F32)

        def fill_ones(i, _):
            rows_v[i, pl.ds(DE, 16)] = one_hot
            return _
        lax.fori_loop(0, CH, fill_ones, None)

        pltpu.sync_copy(z_hbm.at[pl.ds(sid * RPT, RPT)],
                        acc.at[pl.ds(sid * RPT, RPT)])

        @pl.when(sid == 0)
        def _():
            pltpu.sync_copy(z_hbm.at[pl.ds(16 * RPT, TAIL)],
                            acc.at[pl.ds(16 * RPT, TAIL)])
        plsc.subcore_barrier()

        def nbody(i, _):
            c = wid + _NW * i

            @pl.when(c < NCHN)
            def _():
                off = c * CH
                pltpu.sync_copy(ids_hbm.at[pl.ds(off, CH)], idx_v)
                pltpu.async_copy(emb_hbm.at[idx_v], grows_v, sem).wait()
                pltpu.sync_copy(grows_v, hid_out.at[pl.ds(off, CH)])
            return _
        lax.fori_loop(0, NCHN_PW, nbody, None)

        ebase = wid * EPW

        def ebody(i, _):
            off = ebase + i * CH
            pltpu.sync_copy(dst_hbm.at[pl.ds(off, CH)], idx_v)
            pltpu.sync_copy(he_hbm.at[pl.ds(off, CH)], he_v)

            def widen(r, _):
                rows_v[r, pl.ds(0, 16)] = he_v[r, :]
                return _
            lax.fori_loop(0, CH, widen, None)
            pltpu.sync_copy(rows_v, acc.at[idx_v], add=True)
            return _
        lax.fori_loop(0, NCHE, ebody, None)
        plsc.subcore_barrier()

        pltpu.sync_copy(acc.at[pl.ds(sid * RPT, RPT)],
                        ed_out.at[cid, pl.ds(sid * RPT, RPT)])

        @pl.when(sid == 0)
        def _():
            pltpu.sync_copy(acc.at[pl.ds(16 * RPT, TAIL)],
                            ed_out.at[cid, pl.ds(16 * RPT, TAIL)])

    return k(embed_pad, node_ids, H_e, dst, zeros_l)


def _sc_gather_scatter(P, src, dst, zeros_l):
    N, DH = P.shape
    E = src.shape[0]
    CH = 40
    EPW = E // _NW
    NCH = EPW // CH
    R = 5
    assert NCH % R == 0 and NCH > 2 * R
    RPT = (N // 16) // 8 * 8
    TAIL = N - 16 * RPT

    mesh = plsc.VectorSubcoreMesh(core_axis_name="c", subcore_axis_name="s")

    @functools.partial(
        pl.kernel,
        out_type=jax.ShapeDtypeStruct((_NC, N, DH), F32),
        mesh=mesh,
        scratch_types=(
            [pltpu.VMEM((CH,), jnp.int32)] * (2 * R)
            + [pltpu.VMEM((CH, DH), F32)] * R
            + [pltpu.VMEM_SHARED((N, DH), F32)]
            + [pltpu.SemaphoreType.DMA] * (3 * R)
        ),
    )
    def k(p_hbm, src_hbm, dst_hbm, z_hbm, out_hbm, *scr):
        sbuf = scr[0:R]
        dbuf = scr[R:2 * R]
        rows = scr[2 * R:3 * R]
        acc = scr[3 * R]
        isem = scr[3 * R + 1:3 * R + 1 + R]
        gsem = scr[3 * R + 1 + R:3 * R + 1 + 2 * R]
        ssem = scr[3 * R + 1 + 2 * R:3 * R + 1 + 3 * R]

        cid = lax.axis_index("c")
        sid = lax.axis_index("s")
        wid = sid * _NC + cid
        base = wid * EPW

        pltpu.sync_copy(z_hbm.at[pl.ds(sid * RPT, RPT)],
                        acc.at[pl.ds(sid * RPT, RPT)])

        @pl.when(sid == 0)
        def _():
            pltpu.sync_copy(z_hbm.at[pl.ds(16 * RPT, TAIL)],
                            acc.at[pl.ds(16 * RPT, TAIL)])
        plsc.subcore_barrier()

        def start_idx(j, b):
            off = base + j * CH
            pltpu.async_copy(src_hbm.at[pl.ds(off, CH)], sbuf[b], isem[b])
            pltpu.async_copy(dst_hbm.at[pl.ds(off, CH)], dbuf[b], isem[b])

        def wait_idx(j, b):
            off = base + j * CH
            pltpu.make_async_copy(src_hbm.at[pl.ds(off, CH)], sbuf[b],
                                  isem[b]).wait()
            pltpu.make_async_copy(dst_hbm.at[pl.ds(off, CH)], dbuf[b],
                                  isem[b]).wait()

        def start_gather(b):
            pltpu.async_copy(p_hbm.at[sbuf[b]], rows[b], gsem[b])

        def wait_gather(b):
            pltpu.make_async_copy(p_hbm.at[sbuf[b]], rows[b], gsem[b]).wait()

        def wait_scatter(b):
            pltpu.make_async_copy(rows[b], acc.at[dbuf[b]], ssem[b]).wait()

        start_idx(0, 0)
        start_idx(1, 1)
        wait_idx(0, 0)
        start_gather(0)

        def outer(g, _):
            for b in range(R):
                kk = R * g + b
                b2 = (b + 2) % R
                issue = kk + 2 < NCH

                @pl.when(jnp.logical_and(issue, kk >= 3))
                def _():
                    wait_scatter(b2)
                    start_idx(kk + 2, b2)

                @pl.when(jnp.logical_and(issue, kk < 3))
                def _():
                    start_idx(kk + 2, b2)

                b1 = (b + 1) % R

                @pl.when(kk + 1 < NCH)
                def _():
                    wait_idx(kk + 1, b1)
                    start_gather(b1)

                wait_gather(b)
                pltpu.async_copy(rows[b], acc.at[dbuf[b]], ssem[b], add=True)
            return _
        lax.fori_loop(0, NCH // R, outer, None)

        for kk in range(NCH - R, NCH):
            wait_scatter(kk % R)
        plsc.subcore_barrier()

        pltpu.sync_copy(acc.at[pl.ds(sid * RPT, RPT)],
                        out_hbm.at[cid, pl.ds(sid * RPT, RPT)])

        @pl.when(sid == 0)
        def _():
            pltpu.sync_copy(acc.at[pl.ds(16 * RPT, TAIL)],
                            out_hbm.at[cid, pl.ds(16 * RPT, TAIL)])

    return k(P, src, dst, zeros_l)


def _sc_seed_gather(H2, seed_idx):
    N, DH = H2.shape
    S = seed_idx.shape[0]
    BPW = S // _NW

    mesh = plsc.VectorSubcoreMesh(core_axis_name="c", subcore_axis_name="s")

    @functools.partial(
        pl.kernel,
        out_type=jax.ShapeDtypeStruct((S, DH), F32),
        mesh=mesh,
        scratch_types=[
            pltpu.VMEM((BPW,), jnp.int32),
            pltpu.VMEM((BPW, DH), F32),
            pltpu.SemaphoreType.DMA,
        ],
    )
    def k(h_hbm, idx_hbm, out_hbm, idx_v, rows_v, sem):
        wid = lax.axis_index("s") * _NC + lax.axis_index("c")
        base = wid * BPW
        pltpu.sync_copy(idx_hbm.at[pl.ds(base, BPW)], idx_v)
        pltpu.async_copy(h_hbm.at[idx_v], rows_v, sem).wait()
        pltpu.sync_copy(rows_v, out_hbm.at[pl.ds(base, BPW)])

    return k(H2, seed_idx)


def kernel(x, input_node_ids, edge_index, edge_attr, seed_ctx_feat,
           seed_lookup_idx, embed_table, W_feat, b_feat, W_edge, b_edge,
           W1, W1s, b1, W2, W2s, b2, W_ctx, b_ctx, W_pred, b_pred):
    N = x.shape[0]
    S = seed_ctx_feat.shape[0]
    DE = edge_attr.shape[1]
    DID = embed_table.shape[1]

    src = edge_index[0].astype(jnp.int32)
    dst = edge_index[1].astype(jnp.int32)
    ids = input_node_ids.astype(jnp.int32)
    sidx = seed_lookup_idx.astype(jnp.int32)
    zeros_l = jnp.zeros((N, LANES), F32)
    embed_pad = jnp.pad(embed_table, ((0, 0), (0, LANES - DID)))

    H_feat = _tc_node_encode(x, W_feat, b_feat, S)
    H_e = _tc_edge_encode(edge_attr, W_edge, b_edge)

    H_id, edP = _sc_encode_pass(embed_pad, ids, H_e, dst, zeros_l)

    P1, base1, egg2 = _tc_layer1_pre(
        H_feat, H_id, edP[0], edP[1], W1, W1s, W2, b1, DID, DE)

    agg1 = _sc_gather_scatter(P1, src, dst, zeros_l)

    P2, base2 = _tc_layer2_pre(
        agg1[0], agg1[1], edP[0], edP[1], base1, egg2, W2, W2s, b2, DE)

    agg2 = _sc_gather_scatter(P2, src, dst, zeros_l)

    H2 = _tc_layer2_post(agg2[0], agg2[1], edP[0], edP[1], base2, DE)

    seed_embeds = _sc_seed_gather(H2, sidx)

    return _tc_predict(seed_embeds, seed_ctx_feat, W_ctx, b_ctx,
                       W_pred, b_pred)
